# Initial kernel scaffold; baseline (speedup 1.0000x reference)
import jax
import jax.numpy as jnp
from jax.experimental import pallas as pl


def kernel(feat, edge_index, position, W_self, b_self, W_spatial, b_spatial, W_neigh, b_neigh, bias):
    raise NotImplementedError("write your pallas kernel here")


# speedup vs baseline: 1.8275x; 1.8275x over previous
import functools

import jax
import jax.numpy as jnp
from jax import lax
from jax.experimental import pallas as pl
from jax.experimental.pallas import tpu as pltpu
from jax.experimental.pallas import tpu_sc as plsc

EPS = 1e-07
NC = 2
NS = 16
LANES = 16
K = 128


def _rsqrt_newton(v):
    i = plsc.bitcast(v, jnp.int32)
    y = plsc.bitcast(jnp.int32(0x5F3759DF) - (i >> 1), jnp.float32)
    for _ in range(3):
        y = y * (1.5 - 0.5 * v * y * y)
    return y


def _sc_coeff_stage(ncr, cpt, e_pad, src_p, dst_p, posx, posy, posz):
    n = posx.shape[0]
    mesh = plsc.VectorSubcoreMesh(core_axis_name="c", subcore_axis_name="s")

    @functools.partial(
        pl.kernel,
        out_type=(
            jax.ShapeDtypeStruct((e_pad,), jnp.float32),
            jax.ShapeDtypeStruct((e_pad,), jnp.float32),
            jax.ShapeDtypeStruct((e_pad,), jnp.float32),
            jax.ShapeDtypeStruct((NC, ncr, K), jnp.float32),
        ),
        mesh=mesh,
        scratch_types=[
            pltpu.VMEM((n,), jnp.float32),
            pltpu.VMEM((n,), jnp.float32),
            pltpu.VMEM((n,), jnp.float32),
            pltpu.VMEM((K,), jnp.int32),
            pltpu.VMEM((K,), jnp.int32),
            pltpu.VMEM((K,), jnp.float32),
            pltpu.VMEM((K,), jnp.float32),
            pltpu.VMEM((K,), jnp.float32),
            pltpu.VMEM((ncr, K), jnp.float32),
            pltpu.VMEM((ncr,), jnp.int32),
            pltpu.VMEM_SHARED((ncr, K), jnp.float32),
        ],
        compiler_params=pltpu.CompilerParams(needs_layout_passes=False),
    )
    def coeff_kernel(src_hbm, dst_hbm, px_hbm, py_hbm, pz_hbm,
                     a_hbm, b_hbm, c_hbm, cnt_hbm,
                     px_v, py_v, pz_v, srcv, dstv, av, bv, cv,
                     cnt_v, rowidx, cnt_sh):
        cid = lax.axis_index("c")
        sid = lax.axis_index("s")
        wid = cid * NS + sid

        pltpu.sync_copy(px_hbm, px_v)
        pltpu.sync_copy(py_hbm, py_v)
        pltpu.sync_copy(pz_hbm, pz_v)

        zero16 = jnp.zeros((LANES,), jnp.float32)
        iota16 = lax.iota(jnp.int32, LANES)

        def zero_cnt(r, _):
            for cc in range(K // LANES):
                cnt_v[r, pl.ds(cc * LANES, LANES)] = zero16
            return 0

        lax.fori_loop(0, ncr, zero_cnt, 0)

        for i in range(ncr // LANES):
            rowidx[pl.ds(i * LANES, LANES)] = iota16 + i * LANES

        @pl.when(sid < ncr // 8)
        def _():
            pltpu.sync_copy(cnt_v.at[pl.ds(0, 8)],
                            cnt_sh.at[pl.ds(sid * 8, 8)])

        plsc.subcore_barrier()

        ebase = wid * (cpt * K)

        def chunk_body(g, _):
            base = ebase + g * K
            pltpu.sync_copy(src_hbm.at[pl.ds(base, K)], srcv)
            pltpu.sync_copy(dst_hbm.at[pl.ds(base, K)], dstv)

            def group_body(q, _):
                qs = pl.ds(q * LANES, LANES)
                si = srcv[qs]
                di = dstv[qs]
                sx = plsc.load_gather(px_v, [si])
                sy = plsc.load_gather(py_v, [si])
                sz = plsc.load_gather(pz_v, [si])
                dx = plsc.load_gather(px_v, [di])
                dy = plsc.load_gather(py_v, [di])
                dz = plsc.load_gather(pz_v, [di])
                rx = dx - sx
                ry = dy - sy
                rz = dz - sz
                v = rx * rx + ry * ry + rz * rz
                norm = v * _rsqrt_newton(v)
                inv = 1.0 / (norm + EPS)
                av[qs] = (rx + 1.0) * inv
                bv[qs] = (ry + 1.0) * inv
                cv[qs] = (rz + 1.0) * inv

                tot = jnp.zeros((LANES,), jnp.float32)
                pc = jnp.zeros((LANES,), jnp.float32)
                one16 = jnp.full((LANES,), 1.0, jnp.float32)
                for m in range(LANES):
                    eq = di == jnp.full((LANES,), di[m])
                    tot = tot + jnp.where(eq, one16, zero16)
                    if m < LANES - 1:
                        pc = pc + jnp.where(eq & (iota16 > m), one16, zero16)
                first = pc == 0.0
                plsc.addupdate_scatter(cnt_v, [di >> 7, di & 127], tot,
                                       mask=first)
                return 0

            lax.fori_loop(0, K // LANES, group_body, 0)
            pltpu.sync_copy(av, a_hbm.at[pl.ds(base, K)])
            pltpu.sync_copy(bv, b_hbm.at[pl.ds(base, K)])
            pltpu.sync_copy(cv, c_hbm.at[pl.ds(base, K)])
            return 0

        lax.fori_loop(0, cpt, chunk_body, 0)

        pltpu.sync_copy(cnt_v, cnt_sh.at[rowidx], add=True)

        plsc.subcore_barrier()

        @pl.when(sid < ncr // 8)
        def _():
            pltpu.sync_copy(cnt_sh.at[pl.ds(sid * 8, 8)],
                            cnt_hbm.at[cid, pl.ds(sid * 8, 8)])

    return coeff_kernel(src_p, dst_p, posx, posy, posz)


def _sc_scatter_stage(n_pad, cpt, feat, src_p, dst_p, a_e, b_e, c_e,
                      wx, wy, wz, bsp):
    rps = n_pad // NS
    mesh = plsc.VectorSubcoreMesh(core_axis_name="c", subcore_axis_name="s")

    @functools.partial(
        pl.kernel,
        out_type=jax.ShapeDtypeStruct((NC, n_pad, K), jnp.float32),
        mesh=mesh,
        scratch_types=[
            pltpu.VMEM((128,), jnp.float32),
            pltpu.VMEM((128,), jnp.float32),
            pltpu.VMEM((128,), jnp.float32),
            pltpu.VMEM((128,), jnp.float32),
            pltpu.VMEM((K,), jnp.int32),
            pltpu.VMEM((K,), jnp.int32),
            pltpu.VMEM((K,), jnp.float32),
            pltpu.VMEM((K,), jnp.float32),
            pltpu.VMEM((K,), jnp.float32),
            pltpu.VMEM((K, 128), jnp.float32),
            pltpu.VMEM_SHARED((n_pad, K), jnp.float32),
            pltpu.SemaphoreType.DMA,
        ],
        compiler_params=pltpu.CompilerParams(needs_layout_passes=False),
    )
    def scatter_kernel(feat_hbm, src_hbm, dst_hbm, a_hbm, b_hbm, c_hbm,
                       wx_hbm, wy_hbm, wz_hbm, bsp_hbm, out_hbm,
                       wx_v, wy_v, wz_v, bsp_v, srcv, dstv, av, bv, cv,
                       rows, accum, gsem):
        cid = lax.axis_index("c")
        sid = lax.axis_index("s")
        wid = cid * NS + sid

        pltpu.sync_copy(wx_hbm, wx_v)
        pltpu.sync_copy(wy_hbm, wy_v)
        pltpu.sync_copy(wz_hbm, wz_v)
        pltpu.sync_copy(bsp_hbm, bsp_v)

        zero16 = jnp.zeros((LANES,), jnp.float32)

        def zero_row(r, _):
            for cc in range(128 // LANES):
                rows[r, pl.ds(cc * LANES, LANES)] = zero16
            return 0

        lax.fori_loop(0, K, zero_row, 0)

        row0 = sid * rps
        nfull = rps // K
        rem = rps - nfull * K
        for i in range(nfull):
            pltpu.sync_copy(rows, accum.at[pl.ds(row0 + i * K, K)])
        if rem:
            pltpu.sync_copy(rows.at[pl.ds(0, rem)],
                            accum.at[pl.ds(row0 + nfull * K, rem)])

        plsc.subcore_barrier()

        ebase = wid * (cpt * K)

        def chunk_body(g, _):
            base = ebase + g * K
            pltpu.sync_copy(src_hbm.at[pl.ds(base, K)], srcv)
            pltpu.sync_copy(dst_hbm.at[pl.ds(base, K)], dstv)
            pltpu.sync_copy(a_hbm.at[pl.ds(base, K)], av)
            pltpu.sync_copy(b_hbm.at[pl.ds(base, K)], bv)
            pltpu.sync_copy(c_hbm.at[pl.ds(base, K)], cv)
            pltpu.async_copy(feat_hbm.at[srcv], rows, gsem).wait()

            def group_body(q, _):
                qs = pl.ds(q * LANES, LANES)
                aq = av[qs]
                bq = bv[qs]
                cq = cv[qs]
                for l in range(LANES):
                    a = jnp.full((LANES,), aq[l])
                    b = jnp.full((LANES,), bq[l])
                    c = jnp.full((LANES,), cq[l])
                    j = q * LANES + l
                    for c8 in range(128 // LANES):
                        s = pl.ds(c8 * LANES, LANES)
                        z = a * wx_v[s] + b * wy_v[s] + c * wz_v[s] + bsp_v[s]
                        e = jnp.maximum(z, 0.01 * z)
                        rows[j, s] = e * rows[j, s]
                return 0

            lax.fori_loop(0, K // LANES, group_body, 0)
            pltpu.sync_copy(rows, accum.at[dstv], add=True)
            return 0

        lax.fori_loop(0, cpt, chunk_body, 0)

        plsc.subcore_barrier()
        pltpu.sync_copy(accum.at[pl.ds(row0, rps)],
                        out_hbm.at[cid, pl.ds(row0, rps)])

    return scatter_kernel(feat, src_p, dst_p, a_e, b_e, c_e, wx, wy, wz, bsp)


def _tc_body(feat_ref, acc_ref, cnt_ref, ws_ref, wn_ref, b3_ref, out_ref):
    summed = acc_ref[0] + acc_ref[1]
    cnt = cnt_ref[0] + cnt_ref[1]
    h_mean = summed / jnp.maximum(cnt, 1.0)
    dn = (((1,), (1,)), ((), ()))
    t = lax.dot_general(feat_ref[...], ws_ref[...], dn,
                        precision=lax.Precision.HIGHEST,
                        preferred_element_type=jnp.float32)
    t = t + lax.dot_general(h_mean, wn_ref[...], dn,
                            precision=lax.Precision.HIGHEST,
                            preferred_element_type=jnp.float32)
    t = t + (b3_ref[0] + b3_ref[1] + b3_ref[2])[None, :]
    out_ref[...] = jnp.maximum(t, 0.01 * t)


def kernel(feat, edge_index, position, W_self, b_self, W_spatial, b_spatial,
           W_neigh, b_neigh, bias):
    n, f = feat.shape
    e = edge_index.shape[1]

    nw = NC * NS
    cpt = -(-e // (nw * K))
    e_pad = nw * cpt * K
    src_p = jnp.concatenate(
        [edge_index[0], jnp.zeros((e_pad - e,), edge_index.dtype)]).astype(jnp.int32)
    dst_p = jnp.concatenate(
        [edge_index[1], jnp.full((e_pad - e,), n, edge_index.dtype)]).astype(jnp.int32)
    n_pad = -(-(n + 1) // K) * K
    ncr = -(-n_pad // K // LANES) * LANES
    posx = position[:, 0]
    posy = position[:, 1]
    posz = position[:, 2]
    wx = W_spatial[:, 0]
    wy = W_spatial[:, 1]
    wz = W_spatial[:, 2]

    a_e, b_e, c_e, cnt = _sc_coeff_stage(ncr, cpt, e_pad, src_p, dst_p,
                                         posx, posy, posz)
    acc = _sc_scatter_stage(n_pad, cpt, feat, src_p, dst_p, a_e, b_e, c_e,
                            wx, wy, wz, b_spatial)
    cnt_col = cnt.reshape(NC, ncr * K)[:, :n].reshape(NC, n, 1)

    b3 = jnp.stack([b_self, b_neigh, bias])
    blk = 1000
    grid = n // blk
    return pl.pallas_call(
        _tc_body,
        grid=(grid,),
        in_specs=[
            pl.BlockSpec((blk, f), lambda i: (i, 0)),
            pl.BlockSpec((NC, blk, f), lambda i: (0, i, 0)),
            pl.BlockSpec((NC, blk, 1), lambda i: (0, i, 0)),
            pl.BlockSpec((f, f), lambda i: (0, 0)),
            pl.BlockSpec((f, f), lambda i: (0, 0)),
            pl.BlockSpec((3, f), lambda i: (0, 0)),
        ],
        out_specs=pl.BlockSpec((blk, f), lambda i: (i, 0)),
        out_shape=jax.ShapeDtypeStruct((n, f), jnp.float32),
    )(feat, acc[:, :n, :], cnt_col, W_self, W_neigh, b3)
